# Initial kernel scaffold; baseline (speedup 1.0000x reference)
import jax
import jax.numpy as jnp
from jax.experimental import pallas as pl


def kernel(feature_u, feature_v, edge_rows, edge_cols, side_feature_u, side_feature_v, W, W_side, bias_u, bias_v, W_cat_u, W_cat_v):
    raise NotImplementedError("write your pallas kernel here")


# speedup vs baseline: 4.1883x; 4.1883x over previous
import functools

import jax
import jax.numpy as jnp
from jax import lax
from jax.experimental import pallas as pl
from jax.experimental.pallas import tpu as pltpu
from jax.experimental.pallas import tpu_sc as plsc

_N = 10000
_F = 256
_HID = 64
_R = 5
_E = 32000
_NT = 16
_NP = 10240
_ROWS_T = _NP // _NT
_ZCH = 16
_CH = 100
_NCH = (_E // _NT) // _CH
_NBUF = 3
_BLK = 1000


def _proj_body(fv_ref, fu_ref, w_ref, o_ref):
    w = w_ref[0]
    o_ref[:, 0:_HID] = jnp.dot(fv_ref[...], w, preferred_element_type=jnp.float32)
    o_ref[:, _HID:2 * _HID] = jnp.dot(fu_ref[...], w, preferred_element_type=jnp.float32)


_proj_call = pl.pallas_call(
    _proj_body,
    grid=(_N // _BLK, _R),
    in_specs=[
        pl.BlockSpec((_BLK, _F), lambda b, r: (b, 0)),
        pl.BlockSpec((_BLK, _F), lambda b, r: (b, 0)),
        pl.BlockSpec((1, _F, _HID), lambda b, r: (r, 0, 0)),
    ],
    out_specs=pl.BlockSpec((_BLK, 2 * _HID), lambda b, r: (r * (_N // _BLK) + b, 0)),
    out_shape=jax.ShapeDtypeStruct((_R * _N, 2 * _HID), jnp.float32),
)


def _final_body(h_ref, sfu_ref, sfv_ref, ws_ref, bu_ref, bv_ref,
                wcu_ref, wcv_ref, eu_ref, ev_ref):
    ws = ws_ref[...]
    su = jnp.maximum(
        jnp.dot(sfu_ref[...], ws, preferred_element_type=jnp.float32) + bu_ref[...], 0.0)
    sv = jnp.maximum(
        jnp.dot(sfv_ref[...], ws, preferred_element_type=jnp.float32) + bv_ref[...], 0.0)
    accu = jnp.dot(su, wcu_ref[_R * _HID:, :], preferred_element_type=jnp.float32)
    accv = jnp.dot(sv, wcv_ref[_R * _HID:, :], preferred_element_type=jnp.float32)
    for r in range(_R):
        hu = jnp.maximum(h_ref[0, r][:, 0:_HID], 0.0)
        hv = jnp.maximum(h_ref[1, r][:, _HID:2 * _HID], 0.0)
        accu = accu + jnp.dot(hu, wcu_ref[r * _HID:(r + 1) * _HID, :],
                              preferred_element_type=jnp.float32)
        accv = accv + jnp.dot(hv, wcv_ref[r * _HID:(r + 1) * _HID, :],
                              preferred_element_type=jnp.float32)
    eu_ref[...] = jnp.maximum(accu, 0.0)
    ev_ref[...] = jnp.maximum(accv, 0.0)


_final_call = pl.pallas_call(
    _final_body,
    grid=(_N // _BLK,),
    in_specs=[
        pl.BlockSpec((2, _R, _BLK, 2 * _HID), lambda b: (0, 0, b, 0)),

        pl.BlockSpec((_BLK, 128), lambda b: (b, 0)),
        pl.BlockSpec((_BLK, 128), lambda b: (b, 0)),
        pl.BlockSpec((128, _HID), lambda b: (0, 0)),
        pl.BlockSpec((1, _HID), lambda b: (0, 0)),
        pl.BlockSpec((1, _HID), lambda b: (0, 0)),
        pl.BlockSpec((_R * _HID + _HID, _HID), lambda b: (0, 0)),
        pl.BlockSpec((_R * _HID + _HID, _HID), lambda b: (0, 0)),
    ],
    out_specs=[
        pl.BlockSpec((_BLK, _HID), lambda b: (b, 0)),
        pl.BlockSpec((_BLK, _HID), lambda b: (b, 0)),
    ],
    out_shape=[
        jax.ShapeDtypeStruct((_N, _HID), jnp.float32),
        jax.ShapeDtypeStruct((_N, _HID), jnp.float32),
    ],
)


_sc_mesh = plsc.VectorSubcoreMesh(core_axis_name="c", subcore_axis_name="s")


@functools.partial(
    pl.kernel,
    mesh=_sc_mesh,
    out_type=jax.ShapeDtypeStruct((2, _R, _NP, 2 * _HID), jnp.float32),
    scratch_types=[
        pltpu.VMEM((_NCH, _CH), jnp.int32),
        pltpu.VMEM((_NCH, _CH), jnp.int32),
        pltpu.VMEM((_CH, 2 * _HID), jnp.float32),
        pltpu.VMEM((_CH, 2 * _HID), jnp.float32),
        pltpu.VMEM((_CH, 2 * _HID), jnp.float32),
        pltpu.VMEM((_ZCH, 2 * _HID), jnp.float32),
        pltpu.VMEM_SHARED((_NP, 2 * _HID), jnp.float32),
        pltpu.SemaphoreType.DMA,
        pltpu.SemaphoreType.DMA,
        pltpu.SemaphoreType.DMA,
        pltpu.SemaphoreType.DMA,
        pltpu.SemaphoreType.DMA,
        pltpu.SemaphoreType.DMA,
        pltpu.SemaphoreType.DMA,
    ],
)
def _sc_aggregate(proj, src_idx, dst_idx, out,
                  idx_s, idx_d, buf0, buf1, buf2, zeros, acc,
                  gsem0, gsem1, gsem2, ssem0, ssem1, ssem2, psem):
    sid = lax.axis_index("s")
    cid = lax.axis_index("c")

    def _zbody(i, carry):
        for j in range(2 * _HID // 16):
            zeros[i, pl.ds(j * 16, 16)] = jnp.zeros((16,), jnp.float32)
        return carry
    lax.fori_loop(0, _ZCH, _zbody, 0)

    def _run(dirn):
        bufs = (buf0, buf1, buf2)
        gsems = (gsem0, gsem1, gsem2)
        ssems = (ssem0, ssem1, ssem2)

        def _job(r, carry):
            nz = _ROWS_T // _ZCH
            for base in range(0, nz, 4):
                pcps = [pltpu.async_copy(
                            zeros,
                            acc.at[pl.ds(sid * _ROWS_T + (base + q) * _ZCH, _ZCH)],
                            psem)
                        for q in range(min(4, nz - base))]
                for cp in pcps:
                    cp.wait()
            pltpu.sync_copy(src_idx.at[dirn, r, sid], idx_s)
            pltpu.sync_copy(dst_idx.at[dirn, r, sid], idx_d)
            plsc.subcore_barrier()
            gcp = [pltpu.async_copy(proj.at[idx_s.at[b]], bufs[b], gsems[b])
                   for b in range(_NBUF)]
            scp = [None] * _NBUF
            for ch in range(_NCH):
                b = ch % _NBUF
                gcp[b].wait()
                scp[b] = pltpu.async_copy(bufs[b], acc.at[idx_d.at[ch]],
                                          ssems[b], add=True)
                p = ch - (_NBUF - 1)
                if p >= 0 and p + _NBUF < _NCH:
                    scp[p % _NBUF].wait()
                    gcp[p % _NBUF] = pltpu.async_copy(
                        proj.at[idx_s.at[p + _NBUF]], bufs[p % _NBUF],
                        gsems[p % _NBUF])
            for ch in range(_NCH - _NBUF, _NCH):
                scp[ch % _NBUF].wait()
            plsc.subcore_barrier()
            pltpu.sync_copy(acc.at[pl.ds(sid * _ROWS_T, _ROWS_T)],
                            out.at[dirn, r, pl.ds(sid * _ROWS_T, _ROWS_T)])
            return carry

        lax.fori_loop(0, _R, _job, 0)

    @pl.when(cid == 0)
    def _():
        _run(0)

    @pl.when(cid == 1)
    def _():
        _run(1)


def kernel(feature_u, feature_v, edge_rows, edge_cols, side_feature_u,
           side_feature_v, W, W_side, bias_u, bias_v, W_cat_u, W_cat_v):
    er = edge_rows.astype(jnp.int32)
    ec = edge_cols.astype(jnp.int32)
    off = (jnp.arange(_R, dtype=jnp.int32) * _N)[:, None]
    src = jnp.stack([ec + off, er + off]).reshape(2, _R, _NT, _NCH, _CH)
    dst = jnp.stack([er, ec]).reshape(2, _R, _NT, _NCH, _CH)

    proj = _proj_call(feature_v, feature_u, W)
    h_raw = _sc_aggregate(proj, src, dst)
    eu, ev = _final_call(h_raw, side_feature_u, side_feature_v, W_side,
                         bias_u.reshape(1, _HID), bias_v.reshape(1, _HID),
                         W_cat_u, W_cat_v)
    return eu, ev


# speedup vs baseline: 4.2934x; 1.0251x over previous
import functools

import jax
import jax.numpy as jnp
from jax import lax
from jax.experimental import pallas as pl
from jax.experimental.pallas import tpu as pltpu
from jax.experimental.pallas import tpu_sc as plsc

_N = 10000
_F = 256
_HID = 64
_R = 5
_E = 32000
_NT = 16
_NP = 10240
_ROWS_T = _NP // _NT
_ZCH = 64
_CH = 125
_NCH = (_E // _NT) // _CH
_NBUF = 2
_BLK = 1000


def _proj_body(fv_ref, fu_ref, w_ref, o_ref):
    w = w_ref[0]
    o_ref[:, 0:_HID] = jnp.dot(fv_ref[...], w, preferred_element_type=jnp.float32)
    o_ref[:, _HID:2 * _HID] = jnp.dot(fu_ref[...], w, preferred_element_type=jnp.float32)


_proj_call = pl.pallas_call(
    _proj_body,
    grid=(_N // _BLK, _R),
    in_specs=[
        pl.BlockSpec((_BLK, _F), lambda b, r: (b, 0)),
        pl.BlockSpec((_BLK, _F), lambda b, r: (b, 0)),
        pl.BlockSpec((1, _F, _HID), lambda b, r: (r, 0, 0)),
    ],
    out_specs=pl.BlockSpec((_BLK, 2 * _HID), lambda b, r: (r * (_N // _BLK) + b, 0)),
    out_shape=jax.ShapeDtypeStruct((_R * _N, 2 * _HID), jnp.float32),
)


def _final_body(h_ref, sfu_ref, sfv_ref, ws_ref, bu_ref, bv_ref,
                wcu_ref, wcv_ref, eu_ref, ev_ref):
    ws = ws_ref[...]
    su = jnp.maximum(
        jnp.dot(sfu_ref[...], ws, preferred_element_type=jnp.float32) + bu_ref[...], 0.0)
    sv = jnp.maximum(
        jnp.dot(sfv_ref[...], ws, preferred_element_type=jnp.float32) + bv_ref[...], 0.0)
    accu = jnp.dot(su, wcu_ref[_R * _HID:, :], preferred_element_type=jnp.float32)
    accv = jnp.dot(sv, wcv_ref[_R * _HID:, :], preferred_element_type=jnp.float32)
    for r in range(_R):
        hu = jnp.maximum(h_ref[0, r][:, 0:_HID], 0.0)
        hv = jnp.maximum(h_ref[1, r][:, _HID:2 * _HID], 0.0)
        accu = accu + jnp.dot(hu, wcu_ref[r * _HID:(r + 1) * _HID, :],
                              preferred_element_type=jnp.float32)
        accv = accv + jnp.dot(hv, wcv_ref[r * _HID:(r + 1) * _HID, :],
                              preferred_element_type=jnp.float32)
    eu_ref[...] = jnp.maximum(accu, 0.0)
    ev_ref[...] = jnp.maximum(accv, 0.0)


_final_call = pl.pallas_call(
    _final_body,
    grid=(_N // _BLK,),
    in_specs=[
        pl.BlockSpec((2, _R, _BLK, 2 * _HID), lambda b: (0, 0, b, 0)),

        pl.BlockSpec((_BLK, 128), lambda b: (b, 0)),
        pl.BlockSpec((_BLK, 128), lambda b: (b, 0)),
        pl.BlockSpec((128, _HID), lambda b: (0, 0)),
        pl.BlockSpec((1, _HID), lambda b: (0, 0)),
        pl.BlockSpec((1, _HID), lambda b: (0, 0)),
        pl.BlockSpec((_R * _HID + _HID, _HID), lambda b: (0, 0)),
        pl.BlockSpec((_R * _HID + _HID, _HID), lambda b: (0, 0)),
    ],
    out_specs=[
        pl.BlockSpec((_BLK, _HID), lambda b: (b, 0)),
        pl.BlockSpec((_BLK, _HID), lambda b: (b, 0)),
    ],
    out_shape=[
        jax.ShapeDtypeStruct((_N, _HID), jnp.float32),
        jax.ShapeDtypeStruct((_N, _HID), jnp.float32),
    ],
)


_sc_mesh = plsc.VectorSubcoreMesh(core_axis_name="c", subcore_axis_name="s")


@functools.partial(
    pl.kernel,
    mesh=_sc_mesh,
    out_type=jax.ShapeDtypeStruct((2, _R, _NP, 2 * _HID), jnp.float32),
    scratch_types=[
        pltpu.VMEM((_NCH, _CH), jnp.int32),
        pltpu.VMEM((_NCH, _CH), jnp.int32),
        pltpu.VMEM((_CH, 2 * _HID), jnp.float32),
        pltpu.VMEM((_CH, 2 * _HID), jnp.float32),
        pltpu.VMEM((_ZCH, 2 * _HID), jnp.float32),
        pltpu.VMEM_SHARED((_NP, 2 * _HID), jnp.float32),
        pltpu.SemaphoreType.DMA,
        pltpu.SemaphoreType.DMA,
        pltpu.SemaphoreType.DMA,
        pltpu.SemaphoreType.DMA,
        pltpu.SemaphoreType.DMA,
    ],
)
def _sc_aggregate(proj, src_idx, dst_idx, out,
                  idx_s, idx_d, buf0, buf1, zeros, acc,
                  gsem0, gsem1, ssem0, ssem1, psem):
    sid = lax.axis_index("s")
    cid = lax.axis_index("c")

    def _zbody(i, carry):
        for j in range(2 * _HID // 16):
            zeros[i, pl.ds(j * 16, 16)] = jnp.zeros((16,), jnp.float32)
        return carry
    lax.fori_loop(0, _ZCH, _zbody, 0)

    def _run(dirn):
        bufs = (buf0, buf1)
        gsems = (gsem0, gsem1)
        ssems = (ssem0, ssem1)

        def _job(r, carry):
            nz = _ROWS_T // _ZCH
            for base in range(0, nz, 4):
                pcps = [pltpu.async_copy(
                            zeros,
                            acc.at[pl.ds(sid * _ROWS_T + (base + q) * _ZCH, _ZCH)],
                            psem)
                        for q in range(min(4, nz - base))]
                for cp in pcps:
                    cp.wait()
            pltpu.sync_copy(src_idx.at[dirn, r, sid], idx_s)
            pltpu.sync_copy(dst_idx.at[dirn, r, sid], idx_d)
            plsc.subcore_barrier()
            gcp = [pltpu.async_copy(proj.at[idx_s.at[b]], bufs[b], gsems[b])
                   for b in range(_NBUF)]
            scp = [None] * _NBUF
            for ch in range(_NCH):
                b = ch % _NBUF
                gcp[b].wait()
                scp[b] = pltpu.async_copy(bufs[b], acc.at[idx_d.at[ch]],
                                          ssems[b], add=True)
                p = ch - (_NBUF - 1)
                if p >= 0 and p + _NBUF < _NCH:
                    scp[p % _NBUF].wait()
                    gcp[p % _NBUF] = pltpu.async_copy(
                        proj.at[idx_s.at[p + _NBUF]], bufs[p % _NBUF],
                        gsems[p % _NBUF])
            for ch in range(_NCH - _NBUF, _NCH):
                scp[ch % _NBUF].wait()
            plsc.subcore_barrier()
            pltpu.sync_copy(acc.at[pl.ds(sid * _ROWS_T, _ROWS_T)],
                            out.at[dirn, r, pl.ds(sid * _ROWS_T, _ROWS_T)])
            return carry

        lax.fori_loop(0, _R, _job, 0)

    @pl.when(cid == 0)
    def _():
        _run(0)

    @pl.when(cid == 1)
    def _():
        _run(1)


def kernel(feature_u, feature_v, edge_rows, edge_cols, side_feature_u,
           side_feature_v, W, W_side, bias_u, bias_v, W_cat_u, W_cat_v):
    er = edge_rows.astype(jnp.int32)
    ec = edge_cols.astype(jnp.int32)
    off = (jnp.arange(_R, dtype=jnp.int32) * _N)[:, None]
    src = jnp.stack([ec + off, er + off]).reshape(2, _R, _NT, _NCH, _CH)
    dst = jnp.stack([er, ec]).reshape(2, _R, _NT, _NCH, _CH)

    proj = _proj_call(feature_v, feature_u, W)
    h_raw = _sc_aggregate(proj, src, dst)
    eu, ev = _final_call(h_raw, side_feature_u, side_feature_v, W_side,
                         bias_u.reshape(1, _HID), bias_v.reshape(1, _HID),
                         W_cat_u, W_cat_v)
    return eu, ev


# speedup vs baseline: 4.6151x; 1.0749x over previous
import functools

import jax
import jax.numpy as jnp
from jax import lax
from jax.experimental import pallas as pl
from jax.experimental.pallas import tpu as pltpu
from jax.experimental.pallas import tpu_sc as plsc

_N = 10000
_F = 256
_HID = 64
_R = 5
_E = 32000
_NT = 16
_NP = 10240
_ROWS_T = _NP // _NT
_ZCH = 64
_CH = 125
_NCH = (_E // _NT) // _CH
_NBUF = 2
_BLK = 1000


def _proj_body(fv_ref, fu_ref, w_ref, o_ref):
    w = w_ref[0]
    o_ref[:, 0:_HID] = jnp.dot(fv_ref[...], w, preferred_element_type=jnp.float32)
    o_ref[:, _HID:2 * _HID] = jnp.dot(fu_ref[...], w, preferred_element_type=jnp.float32)


_proj_call = pl.pallas_call(
    _proj_body,
    grid=(_N // _BLK, _R),
    in_specs=[
        pl.BlockSpec((_BLK, _F), lambda b, r: (b, 0)),
        pl.BlockSpec((_BLK, _F), lambda b, r: (b, 0)),
        pl.BlockSpec((1, _F, _HID), lambda b, r: (r, 0, 0)),
    ],
    out_specs=pl.BlockSpec((_BLK, 2 * _HID), lambda b, r: (r * (_N // _BLK) + b, 0)),
    out_shape=jax.ShapeDtypeStruct((_R * _N, 2 * _HID), jnp.float32),
)


def _final_body(h_ref, sfu_ref, sfv_ref, ws_ref, bu_ref, bv_ref,
                wcu_ref, wcv_ref, eu_ref, ev_ref):
    ws = ws_ref[...]
    su = jnp.maximum(
        jnp.dot(sfu_ref[...], ws, preferred_element_type=jnp.float32) + bu_ref[...], 0.0)
    sv = jnp.maximum(
        jnp.dot(sfv_ref[...], ws, preferred_element_type=jnp.float32) + bv_ref[...], 0.0)
    accu = jnp.dot(su, wcu_ref[_R * _HID:, :], preferred_element_type=jnp.float32)
    accv = jnp.dot(sv, wcv_ref[_R * _HID:, :], preferred_element_type=jnp.float32)
    for r in range(_R):
        hu = jnp.maximum(h_ref[0, r][:, 0:_HID], 0.0)
        hv = jnp.maximum(h_ref[1, r][:, _HID:2 * _HID], 0.0)
        accu = accu + jnp.dot(hu, wcu_ref[r * _HID:(r + 1) * _HID, :],
                              preferred_element_type=jnp.float32)
        accv = accv + jnp.dot(hv, wcv_ref[r * _HID:(r + 1) * _HID, :],
                              preferred_element_type=jnp.float32)
    eu_ref[...] = jnp.maximum(accu, 0.0)
    ev_ref[...] = jnp.maximum(accv, 0.0)


_final_call = pl.pallas_call(
    _final_body,
    grid=(_N // _BLK,),
    in_specs=[
        pl.BlockSpec((2, _R, _BLK, 2 * _HID), lambda b: (0, 0, b, 0)),

        pl.BlockSpec((_BLK, 128), lambda b: (b, 0)),
        pl.BlockSpec((_BLK, 128), lambda b: (b, 0)),
        pl.BlockSpec((128, _HID), lambda b: (0, 0)),
        pl.BlockSpec((1, _HID), lambda b: (0, 0)),
        pl.BlockSpec((1, _HID), lambda b: (0, 0)),
        pl.BlockSpec((_R * _HID + _HID, _HID), lambda b: (0, 0)),
        pl.BlockSpec((_R * _HID + _HID, _HID), lambda b: (0, 0)),
    ],
    out_specs=[
        pl.BlockSpec((_BLK, _HID), lambda b: (b, 0)),
        pl.BlockSpec((_BLK, _HID), lambda b: (b, 0)),
    ],
    out_shape=[
        jax.ShapeDtypeStruct((_N, _HID), jnp.float32),
        jax.ShapeDtypeStruct((_N, _HID), jnp.float32),
    ],
)


_sc_mesh = plsc.VectorSubcoreMesh(core_axis_name="c", subcore_axis_name="s")


@functools.partial(
    pl.kernel,
    mesh=_sc_mesh,
    out_type=jax.ShapeDtypeStruct((2, _R, _NP, 2 * _HID), jnp.float32),
    scratch_types=[
        pltpu.VMEM((_NCH, _CH), jnp.int32),
        pltpu.VMEM((_NCH, _CH), jnp.int32),
        pltpu.VMEM((_CH, 2 * _HID), jnp.float32),
        pltpu.VMEM((_CH, 2 * _HID), jnp.float32),
        pltpu.VMEM((_ZCH, 2 * _HID), jnp.float32),
        pltpu.VMEM_SHARED((_NP, 2 * _HID), jnp.float32),
        pltpu.SemaphoreType.DMA,
        pltpu.SemaphoreType.DMA,
        pltpu.SemaphoreType.DMA,
    ],
)
def _sc_aggregate(proj, src_idx, dst_idx, out,
                  idx_s, idx_d, buf0, buf1, zeros, acc,
                  gsem0, gsem1, psem):
    sid = lax.axis_index("s")
    cid = lax.axis_index("c")

    def _zbody(i, carry):
        for j in range(2 * _HID // 16):
            zeros[i, pl.ds(j * 16, 16)] = jnp.zeros((16,), jnp.float32)
        return carry
    lax.fori_loop(0, _ZCH, _zbody, 0)

    def _run(dirn):
        bufs = (buf0, buf1)
        gsems = (gsem0, gsem1)

        def _job(r, carry):
            for q in range(_ROWS_T // _ZCH):
                pltpu.sync_copy(zeros, acc.at[pl.ds(sid * _ROWS_T + q * _ZCH, _ZCH)])
            pltpu.sync_copy(src_idx.at[dirn, r, sid], idx_s)
            pltpu.sync_copy(dst_idx.at[dirn, r, sid], idx_d)
            plsc.subcore_barrier()
            cps = [pltpu.async_copy(proj.at[idx_s.at[0]], bufs[0], gsems[0]), None]
            for ch in range(_NCH):
                nx = ch + 1
                if nx < _NCH:
                    cps[nx % _NBUF] = pltpu.async_copy(
                        proj.at[idx_s.at[nx]], bufs[nx % _NBUF],
                        gsems[nx % _NBUF])
                cps[ch % _NBUF].wait()
                pltpu.sync_copy(bufs[ch % _NBUF], acc.at[idx_d.at[ch]], add=True)
            plsc.subcore_barrier()
            pltpu.sync_copy(acc.at[pl.ds(sid * _ROWS_T, _ROWS_T)],
                            out.at[dirn, r, pl.ds(sid * _ROWS_T, _ROWS_T)])
            return carry

        lax.fori_loop(0, _R, _job, 0)

    @pl.when(cid == 0)
    def _():
        _run(0)

    @pl.when(cid == 1)
    def _():
        _run(1)


def kernel(feature_u, feature_v, edge_rows, edge_cols, side_feature_u,
           side_feature_v, W, W_side, bias_u, bias_v, W_cat_u, W_cat_v):
    er = edge_rows.astype(jnp.int32)
    ec = edge_cols.astype(jnp.int32)
    off = (jnp.arange(_R, dtype=jnp.int32) * _N)[:, None]
    src = jnp.stack([ec + off, er + off]).reshape(2, _R, _NT, _NCH, _CH)
    dst = jnp.stack([er, ec]).reshape(2, _R, _NT, _NCH, _CH)

    proj = _proj_call(feature_v, feature_u, W)
    h_raw = _sc_aggregate(proj, src, dst)
    eu, ev = _final_call(h_raw, side_feature_u, side_feature_v, W_side,
                         bias_u.reshape(1, _HID), bias_v.reshape(1, _HID),
                         W_cat_u, W_cat_v)
    return eu, ev


# speedup vs baseline: 4.7713x; 1.0338x over previous
import functools

import jax
import jax.numpy as jnp
from jax import lax
from jax.experimental import pallas as pl
from jax.experimental.pallas import tpu as pltpu
from jax.experimental.pallas import tpu_sc as plsc

_N = 10000
_F = 256
_HID = 64
_R = 5
_E = 32000
_NT = 16
_NP = 10240
_ROWS_T = _NP // _NT
_ZCH = 64
_CH = 125
_NCH = (_E // _NT) // _CH
_NBUF = 2
_BLK = 1000


def _proj_body(fv_ref, fu_ref, w_ref, o_ref):
    w = w_ref[0]
    o_ref[:, 0:_HID] = jnp.dot(fv_ref[...], w, preferred_element_type=jnp.float32)
    o_ref[:, _HID:2 * _HID] = jnp.dot(fu_ref[...], w, preferred_element_type=jnp.float32)


_proj_call = pl.pallas_call(
    _proj_body,
    grid=(_N // _BLK, _R),
    in_specs=[
        pl.BlockSpec((_BLK, _F), lambda b, r: (b, 0)),
        pl.BlockSpec((_BLK, _F), lambda b, r: (b, 0)),
        pl.BlockSpec((1, _F, _HID), lambda b, r: (r, 0, 0)),
    ],
    out_specs=pl.BlockSpec((_BLK, 2 * _HID), lambda b, r: (r * (_N // _BLK) + b, 0)),
    out_shape=jax.ShapeDtypeStruct((_R * _N, 2 * _HID), jnp.float32),
)


def _final_body(h_ref, sfu_ref, sfv_ref, ws_ref, bu_ref, bv_ref,
                wcu_ref, wcv_ref, eu_ref, ev_ref):
    ws = ws_ref[...]
    su = jnp.maximum(
        jnp.dot(sfu_ref[...], ws, preferred_element_type=jnp.float32) + bu_ref[...], 0.0)
    sv = jnp.maximum(
        jnp.dot(sfv_ref[...], ws, preferred_element_type=jnp.float32) + bv_ref[...], 0.0)
    accu = jnp.dot(su, wcu_ref[_R * _HID:, :], preferred_element_type=jnp.float32)
    accv = jnp.dot(sv, wcv_ref[_R * _HID:, :], preferred_element_type=jnp.float32)
    for r in range(_R):
        hu = jnp.maximum(h_ref[0, r][:, 0:_HID], 0.0)
        hv = jnp.maximum(h_ref[1, r][:, _HID:2 * _HID], 0.0)
        accu = accu + jnp.dot(hu, wcu_ref[r * _HID:(r + 1) * _HID, :],
                              preferred_element_type=jnp.float32)
        accv = accv + jnp.dot(hv, wcv_ref[r * _HID:(r + 1) * _HID, :],
                              preferred_element_type=jnp.float32)
    eu_ref[...] = jnp.maximum(accu, 0.0)
    ev_ref[...] = jnp.maximum(accv, 0.0)


_final_call = pl.pallas_call(
    _final_body,
    grid=(_N // _BLK,),
    in_specs=[
        pl.BlockSpec((2, _R, _BLK, 2 * _HID), lambda b: (0, 0, b, 0)),

        pl.BlockSpec((_BLK, 128), lambda b: (b, 0)),
        pl.BlockSpec((_BLK, 128), lambda b: (b, 0)),
        pl.BlockSpec((128, _HID), lambda b: (0, 0)),
        pl.BlockSpec((1, _HID), lambda b: (0, 0)),
        pl.BlockSpec((1, _HID), lambda b: (0, 0)),
        pl.BlockSpec((_R * _HID + _HID, _HID), lambda b: (0, 0)),
        pl.BlockSpec((_R * _HID + _HID, _HID), lambda b: (0, 0)),
    ],
    out_specs=[
        pl.BlockSpec((_BLK, _HID), lambda b: (b, 0)),
        pl.BlockSpec((_BLK, _HID), lambda b: (b, 0)),
    ],
    out_shape=[
        jax.ShapeDtypeStruct((_N, _HID), jnp.float32),
        jax.ShapeDtypeStruct((_N, _HID), jnp.float32),
    ],
)


_sc_mesh = plsc.VectorSubcoreMesh(core_axis_name="c", subcore_axis_name="s")


@functools.partial(
    pl.kernel,
    mesh=_sc_mesh,
    out_type=jax.ShapeDtypeStruct((2, _R, _NP, 2 * _HID), jnp.float32),
    scratch_types=[
        pltpu.VMEM((_NCH, _CH), jnp.int32),
        pltpu.VMEM((_NCH, _CH), jnp.int32),
        pltpu.VMEM((_CH, 2 * _HID), jnp.float32),
        pltpu.VMEM((_CH, 2 * _HID), jnp.float32),
        pltpu.VMEM((_ZCH, 2 * _HID), jnp.float32),
        pltpu.VMEM_SHARED((_NP, 2 * _HID), jnp.float32),
        pltpu.SemaphoreType.DMA,
        pltpu.SemaphoreType.DMA,
        pltpu.SemaphoreType.DMA,
    ],
)
def _sc_aggregate(proj, src_idx, dst_idx, out,
                  idx_s, idx_d, buf0, buf1, zeros, acc,
                  gsem0, gsem1, psem):
    sid = lax.axis_index("s")
    cid = lax.axis_index("c")

    def _zbody(i, carry):
        for j in range(2 * _HID // 16):
            zeros[i, pl.ds(j * 16, 16)] = jnp.zeros((16,), jnp.float32)
        return carry
    lax.fori_loop(0, _ZCH, _zbody, 0)

    def _run(dirn):
        bufs = (buf0, buf1)
        gsems = (gsem0, gsem1)

        def _job(r, carry):
            pltpu.sync_copy(src_idx.at[dirn, r, sid], idx_s)
            pltpu.sync_copy(dst_idx.at[dirn, r, sid], idx_d)
            cps = [pltpu.async_copy(proj.at[idx_s.at[b]], bufs[b], gsems[b])
                   for b in range(_NBUF)]
            for q in range(_ROWS_T // _ZCH):
                pltpu.sync_copy(zeros, acc.at[pl.ds(sid * _ROWS_T + q * _ZCH, _ZCH)])
            plsc.subcore_barrier()
            for ch in range(_NCH):
                nx = ch + 1
                if _NBUF <= nx < _NCH:
                    cps[nx % _NBUF] = pltpu.async_copy(
                        proj.at[idx_s.at[nx]], bufs[nx % _NBUF],
                        gsems[nx % _NBUF])
                cps[ch % _NBUF].wait()
                pltpu.sync_copy(bufs[ch % _NBUF], acc.at[idx_d.at[ch]], add=True)
            plsc.subcore_barrier()
            pltpu.sync_copy(acc.at[pl.ds(sid * _ROWS_T, _ROWS_T)],
                            out.at[dirn, r, pl.ds(sid * _ROWS_T, _ROWS_T)])
            return carry

        lax.fori_loop(0, _R, _job, 0)

    @pl.when(cid == 0)
    def _():
        _run(0)

    @pl.when(cid == 1)
    def _():
        _run(1)


def kernel(feature_u, feature_v, edge_rows, edge_cols, side_feature_u,
           side_feature_v, W, W_side, bias_u, bias_v, W_cat_u, W_cat_v):
    er = edge_rows.astype(jnp.int32)
    ec = edge_cols.astype(jnp.int32)
    off = (jnp.arange(_R, dtype=jnp.int32) * _N)[:, None]
    src = jnp.stack([ec + off, er + off]).reshape(2, _R, _NT, _NCH, _CH)
    dst = jnp.stack([er, ec]).reshape(2, _R, _NT, _NCH, _CH)

    proj = _proj_call(feature_v, feature_u, W)
    h_raw = _sc_aggregate(proj, src, dst)
    eu, ev = _final_call(h_raw, side_feature_u, side_feature_v, W_side,
                         bias_u.reshape(1, _HID), bias_v.reshape(1, _HID),
                         W_cat_u, W_cat_v)
    return eu, ev


# speedup vs baseline: 4.8942x; 1.0258x over previous
import functools

import jax
import jax.numpy as jnp
from jax import lax
from jax.experimental import pallas as pl
from jax.experimental.pallas import tpu as pltpu
from jax.experimental.pallas import tpu_sc as plsc

_N = 10000
_F = 256
_HID = 64
_R = 5
_E = 32000
_NT = 16
_NP = 10240
_ROWS_T = _NP // _NT
_ZCH = 64
_CH = 125
_NCH = (_E // _NT) // _CH
_NBUF = 2
_BLK = 1000


def _proj_body(fv_ref, fu_ref, w_ref, o_ref):
    w = w_ref[0]
    o_ref[:, 0:_HID] = jnp.dot(fv_ref[...], w, preferred_element_type=jnp.float32)
    o_ref[:, _HID:2 * _HID] = jnp.dot(fu_ref[...], w, preferred_element_type=jnp.float32)


_proj_call = pl.pallas_call(
    _proj_body,
    grid=(_N // _BLK, _R),
    in_specs=[
        pl.BlockSpec((_BLK, _F), lambda b, r: (b, 0)),
        pl.BlockSpec((_BLK, _F), lambda b, r: (b, 0)),
        pl.BlockSpec((1, _F, _HID), lambda b, r: (r, 0, 0)),
    ],
    out_specs=pl.BlockSpec((_BLK, 2 * _HID), lambda b, r: (r * (_N // _BLK) + b, 0)),
    out_shape=jax.ShapeDtypeStruct((_R * _N, 2 * _HID), jnp.float32),
)


def _final_body(h_ref, sfu_ref, sfv_ref, ws_ref, bu_ref, bv_ref,
                wcu_ref, wcv_ref, eu_ref, ev_ref):
    ws = ws_ref[...]
    su = jnp.maximum(
        jnp.dot(sfu_ref[...], ws, preferred_element_type=jnp.float32) + bu_ref[...], 0.0)
    sv = jnp.maximum(
        jnp.dot(sfv_ref[...], ws, preferred_element_type=jnp.float32) + bv_ref[...], 0.0)
    accu = jnp.dot(su, wcu_ref[_R * _HID:, :], preferred_element_type=jnp.float32)
    accv = jnp.dot(sv, wcv_ref[_R * _HID:, :], preferred_element_type=jnp.float32)
    for r in range(_R):
        hu = jnp.maximum(h_ref[0, r][:, 0:_HID], 0.0)
        hv = jnp.maximum(h_ref[1, r][:, _HID:2 * _HID], 0.0)
        accu = accu + jnp.dot(hu, wcu_ref[r * _HID:(r + 1) * _HID, :],
                              preferred_element_type=jnp.float32)
        accv = accv + jnp.dot(hv, wcv_ref[r * _HID:(r + 1) * _HID, :],
                              preferred_element_type=jnp.float32)
    eu_ref[...] = jnp.maximum(accu, 0.0)
    ev_ref[...] = jnp.maximum(accv, 0.0)


_final_call = pl.pallas_call(
    _final_body,
    grid=(_N // _BLK,),
    in_specs=[
        pl.BlockSpec((2, _R, _BLK, 2 * _HID), lambda b: (0, 0, b, 0)),

        pl.BlockSpec((_BLK, 128), lambda b: (b, 0)),
        pl.BlockSpec((_BLK, 128), lambda b: (b, 0)),
        pl.BlockSpec((128, _HID), lambda b: (0, 0)),
        pl.BlockSpec((1, _HID), lambda b: (0, 0)),
        pl.BlockSpec((1, _HID), lambda b: (0, 0)),
        pl.BlockSpec((_R * _HID + _HID, _HID), lambda b: (0, 0)),
        pl.BlockSpec((_R * _HID + _HID, _HID), lambda b: (0, 0)),
    ],
    out_specs=[
        pl.BlockSpec((_BLK, _HID), lambda b: (b, 0)),
        pl.BlockSpec((_BLK, _HID), lambda b: (b, 0)),
    ],
    out_shape=[
        jax.ShapeDtypeStruct((_N, _HID), jnp.float32),
        jax.ShapeDtypeStruct((_N, _HID), jnp.float32),
    ],
)


_sc_mesh = plsc.VectorSubcoreMesh(core_axis_name="c", subcore_axis_name="s")


@functools.partial(
    pl.kernel,
    mesh=_sc_mesh,
    out_type=jax.ShapeDtypeStruct((2, _R, _NP, 2 * _HID), jnp.float32),
    scratch_types=[
        pltpu.VMEM((_NCH, _CH), jnp.int32),
        pltpu.VMEM((_NCH, _CH), jnp.int32),
        pltpu.VMEM((_CH, 2 * _HID), jnp.float32),
        pltpu.VMEM((_CH, 2 * _HID), jnp.float32),
        pltpu.VMEM((_ZCH, 2 * _HID), jnp.float32),
        pltpu.VMEM_SHARED((_NP, 2 * _HID), jnp.float32),
        pltpu.SemaphoreType.DMA,
        pltpu.SemaphoreType.DMA,
        pltpu.SemaphoreType.DMA,
    ],
)
def _sc_aggregate(proj, src_idx, dst_idx, out,
                  idx_s, idx_d, buf0, buf1, zeros, acc,
                  gsem0, gsem1, psem):
    sid = lax.axis_index("s")
    cid = lax.axis_index("c")

    def _zbody(i, carry):
        for j in range(2 * _HID // 16):
            zeros[i, pl.ds(j * 16, 16)] = jnp.zeros((16,), jnp.float32)
        return carry
    lax.fori_loop(0, _ZCH, _zbody, 0)

    def _run(dirn):
        bufs = (buf0, buf1)
        gsems = (gsem0, gsem1)

        def _job(r, carry):
            pltpu.sync_copy(src_idx.at[dirn, r, sid], idx_s)
            pltpu.sync_copy(dst_idx.at[dirn, r, sid], idx_d)
            cps = [pltpu.async_copy(proj.at[idx_s.at[b]], bufs[b], gsems[b])
                   for b in range(_NBUF)]

            @pl.when(r > 0)
            def _():
                pltpu.make_async_copy(
                    out.at[dirn, 0, pl.ds(sid * _ROWS_T, _ROWS_T)],
                    acc.at[pl.ds(sid * _ROWS_T, _ROWS_T)], psem).wait()

            for q in range(_ROWS_T // _ZCH):
                pltpu.sync_copy(zeros, acc.at[pl.ds(sid * _ROWS_T + q * _ZCH, _ZCH)])
            plsc.subcore_barrier()
            for ch in range(_NCH):
                nx = ch + 1
                if _NBUF <= nx < _NCH:
                    cps[nx % _NBUF] = pltpu.async_copy(
                        proj.at[idx_s.at[nx]], bufs[nx % _NBUF],
                        gsems[nx % _NBUF])
                cps[ch % _NBUF].wait()
                pltpu.sync_copy(bufs[ch % _NBUF], acc.at[idx_d.at[ch]], add=True)
            plsc.subcore_barrier()
            pltpu.async_copy(acc.at[pl.ds(sid * _ROWS_T, _ROWS_T)],
                             out.at[dirn, r, pl.ds(sid * _ROWS_T, _ROWS_T)], psem)
            return carry

        lax.fori_loop(0, _R, _job, 0)
        pltpu.make_async_copy(
            out.at[dirn, 0, pl.ds(sid * _ROWS_T, _ROWS_T)],
            acc.at[pl.ds(sid * _ROWS_T, _ROWS_T)], psem).wait()

    @pl.when(cid == 0)
    def _():
        _run(0)

    @pl.when(cid == 1)
    def _():
        _run(1)


def kernel(feature_u, feature_v, edge_rows, edge_cols, side_feature_u,
           side_feature_v, W, W_side, bias_u, bias_v, W_cat_u, W_cat_v):
    er = edge_rows.astype(jnp.int32)
    ec = edge_cols.astype(jnp.int32)
    off = (jnp.arange(_R, dtype=jnp.int32) * _N)[:, None]
    src = jnp.stack([ec + off, er + off]).reshape(2, _R, _NT, _NCH, _CH)
    dst = jnp.stack([er, ec]).reshape(2, _R, _NT, _NCH, _CH)

    proj = _proj_call(feature_v, feature_u, W)
    h_raw = _sc_aggregate(proj, src, dst)
    eu, ev = _final_call(h_raw, side_feature_u, side_feature_v, W_side,
                         bias_u.reshape(1, _HID), bias_v.reshape(1, _HID),
                         W_cat_u, W_cat_v)
    return eu, ev


# speedup vs baseline: 5.3258x; 1.0882x over previous
import functools

import jax
import jax.numpy as jnp
from jax import lax
from jax.experimental import pallas as pl
from jax.experimental.pallas import tpu as pltpu
from jax.experimental.pallas import tpu_sc as plsc

_N = 10000
_F = 256
_HID = 64
_R = 5
_E = 32000
_NT = 16
_NP = 10240
_ROWS_T = _NP // _NT
_ZCH = 64
_CH = 125
_NCH = (_E // _NT) // _CH
_NBUF = 2
_BLK = 2000


def _proj_body(fv_ref, fu_ref, w_ref, o_ref):
    w = w_ref[0]
    o_ref[:, 0:_HID] = jnp.dot(fv_ref[...], w, preferred_element_type=jnp.float32)
    o_ref[:, _HID:2 * _HID] = jnp.dot(fu_ref[...], w, preferred_element_type=jnp.float32)


_proj_call = pl.pallas_call(
    _proj_body,
    grid=(_N // _BLK, _R),
    in_specs=[
        pl.BlockSpec((_BLK, _F), lambda b, r: (b, 0)),
        pl.BlockSpec((_BLK, _F), lambda b, r: (b, 0)),
        pl.BlockSpec((1, _F, _HID), lambda b, r: (r, 0, 0)),
    ],
    out_specs=pl.BlockSpec((_BLK, 2 * _HID), lambda b, r: (r * (_N // _BLK) + b, 0)),
    out_shape=jax.ShapeDtypeStruct((_R * _N, 2 * _HID), jnp.float32),
)


def _final_body(h_ref, sfu_ref, sfv_ref, ws_ref, bu_ref, bv_ref,
                wcu_ref, wcv_ref, eu_ref, ev_ref):
    ws = ws_ref[...]
    su = jnp.maximum(
        jnp.dot(sfu_ref[...], ws, preferred_element_type=jnp.float32) + bu_ref[...], 0.0)
    sv = jnp.maximum(
        jnp.dot(sfv_ref[...], ws, preferred_element_type=jnp.float32) + bv_ref[...], 0.0)
    accu = jnp.dot(su, wcu_ref[_R * _HID:, :], preferred_element_type=jnp.float32)
    accv = jnp.dot(sv, wcv_ref[_R * _HID:, :], preferred_element_type=jnp.float32)
    for r in range(_R):
        hu = jnp.maximum(h_ref[0, r][:, 0:_HID], 0.0)
        hv = jnp.maximum(h_ref[1, r][:, _HID:2 * _HID], 0.0)
        accu = accu + jnp.dot(hu, wcu_ref[r * _HID:(r + 1) * _HID, :],
                              preferred_element_type=jnp.float32)
        accv = accv + jnp.dot(hv, wcv_ref[r * _HID:(r + 1) * _HID, :],
                              preferred_element_type=jnp.float32)
    eu_ref[...] = jnp.maximum(accu, 0.0)
    ev_ref[...] = jnp.maximum(accv, 0.0)


_final_call = pl.pallas_call(
    _final_body,
    grid=(_N // _BLK,),
    in_specs=[
        pl.BlockSpec((2, _R, _BLK, 2 * _HID), lambda b: (0, 0, b, 0)),

        pl.BlockSpec((_BLK, 128), lambda b: (b, 0)),
        pl.BlockSpec((_BLK, 128), lambda b: (b, 0)),
        pl.BlockSpec((128, _HID), lambda b: (0, 0)),
        pl.BlockSpec((1, _HID), lambda b: (0, 0)),
        pl.BlockSpec((1, _HID), lambda b: (0, 0)),
        pl.BlockSpec((_R * _HID + _HID, _HID), lambda b: (0, 0)),
        pl.BlockSpec((_R * _HID + _HID, _HID), lambda b: (0, 0)),
    ],
    out_specs=[
        pl.BlockSpec((_BLK, _HID), lambda b: (b, 0)),
        pl.BlockSpec((_BLK, _HID), lambda b: (b, 0)),
    ],
    out_shape=[
        jax.ShapeDtypeStruct((_N, _HID), jnp.float32),
        jax.ShapeDtypeStruct((_N, _HID), jnp.float32),
    ],
)


_sc_mesh = plsc.VectorSubcoreMesh(core_axis_name="c", subcore_axis_name="s")


@functools.partial(
    pl.kernel,
    mesh=_sc_mesh,
    out_type=jax.ShapeDtypeStruct((2, _R, _NP, 2 * _HID), jnp.float32),
    scratch_types=[
        pltpu.VMEM((_NCH, _CH), jnp.int32),
        pltpu.VMEM((_NCH, _CH), jnp.int32),
        pltpu.VMEM((_CH, 2 * _HID), jnp.float32),
        pltpu.VMEM((_CH, 2 * _HID), jnp.float32),
        pltpu.VMEM((_ZCH, 2 * _HID), jnp.float32),
        pltpu.VMEM_SHARED((_NP, 2 * _HID), jnp.float32),
        pltpu.SemaphoreType.DMA,
        pltpu.SemaphoreType.DMA,
        pltpu.SemaphoreType.DMA,
    ],
)
def _sc_aggregate(proj, src_idx, dst_idx, out,
                  idx_s, idx_d, buf0, buf1, zeros, acc,
                  gsem0, gsem1, psem):
    sid = lax.axis_index("s")
    cid = lax.axis_index("c")

    def _zbody(i, carry):
        for j in range(2 * _HID // 16):
            zeros[i, pl.ds(j * 16, 16)] = jnp.zeros((16,), jnp.float32)
        return carry
    lax.fori_loop(0, _ZCH, _zbody, 0)

    def _run(dirn):
        bufs = (buf0, buf1)
        gsems = (gsem0, gsem1)

        def _job(r, carry):
            pltpu.sync_copy(src_idx.at[dirn, r, sid], idx_s)
            pltpu.sync_copy(dst_idx.at[dirn, r, sid], idx_d)
            cps = [pltpu.async_copy(proj.at[idx_s.at[b]], bufs[b], gsems[b])
                   for b in range(_NBUF)]

            @pl.when(r > 0)
            def _():
                pltpu.make_async_copy(
                    out.at[dirn, 0, pl.ds(sid * _ROWS_T, _ROWS_T)],
                    acc.at[pl.ds(sid * _ROWS_T, _ROWS_T)], psem).wait()

            for q in range(_ROWS_T // _ZCH):
                pltpu.sync_copy(zeros, acc.at[pl.ds(sid * _ROWS_T + q * _ZCH, _ZCH)])
            plsc.subcore_barrier()
            for ch in range(_NCH):
                nx = ch + 1
                if _NBUF <= nx < _NCH:
                    cps[nx % _NBUF] = pltpu.async_copy(
                        proj.at[idx_s.at[nx]], bufs[nx % _NBUF],
                        gsems[nx % _NBUF])
                cps[ch % _NBUF].wait()
                pltpu.sync_copy(bufs[ch % _NBUF], acc.at[idx_d.at[ch]], add=True)
            plsc.subcore_barrier()
            pltpu.async_copy(acc.at[pl.ds(sid * _ROWS_T, _ROWS_T)],
                             out.at[dirn, r, pl.ds(sid * _ROWS_T, _ROWS_T)], psem)
            return carry

        lax.fori_loop(0, _R, _job, 0)
        pltpu.make_async_copy(
            out.at[dirn, 0, pl.ds(sid * _ROWS_T, _ROWS_T)],
            acc.at[pl.ds(sid * _ROWS_T, _ROWS_T)], psem).wait()

    @pl.when(cid == 0)
    def _():
        _run(0)

    @pl.when(cid == 1)
    def _():
        _run(1)


def kernel(feature_u, feature_v, edge_rows, edge_cols, side_feature_u,
           side_feature_v, W, W_side, bias_u, bias_v, W_cat_u, W_cat_v):
    er = edge_rows.astype(jnp.int32)
    ec = edge_cols.astype(jnp.int32)
    off = (jnp.arange(_R, dtype=jnp.int32) * _N)[:, None]
    src = jnp.stack([ec + off, er + off]).reshape(2, _R, _NT, _NCH, _CH)
    dst = jnp.stack([er, ec]).reshape(2, _R, _NT, _NCH, _CH)

    proj = _proj_call(feature_v, feature_u, W)
    h_raw = _sc_aggregate(proj, src, dst)
    eu, ev = _final_call(h_raw, side_feature_u, side_feature_v, W_side,
                         bias_u.reshape(1, _HID), bias_v.reshape(1, _HID),
                         W_cat_u, W_cat_v)
    return eu, ev
